# Initial kernel scaffold; baseline (speedup 1.0000x reference)
import jax
import jax.numpy as jnp
from jax.experimental import pallas as pl
from jax.experimental.pallas import tpu as pltpu

NE = 64
K = 8
D = 2048
T = 16384
R = 512
GRID = T // R
ALPHA = 0.001


def _gate_body(x_ref, w_ref, tw_ref, ti_ref, aux_ref, pi_ref, cnt_ref):
    step = pl.program_id(0)
    logits = jax.lax.dot_general(
        w_ref[...], x_ref[...], (((1,), (1,)), ((), ())),
        preferred_element_type=jnp.float32)
    m = jnp.max(logits, axis=0, keepdims=True)
    e = jnp.exp(logits - m)
    s = jnp.sum(e, axis=0, keepdims=True)
    scores = e / s

    sub_iota = jax.lax.broadcasted_iota(jnp.int32, (NE, R), 0)
    work = scores
    tws, tis = [], []
    onehot_sum = jnp.zeros((NE, R), jnp.float32)
    for _ in range(K):
        mk = jnp.max(work, axis=0, keepdims=True)
        eq = work == mk
        idx = jnp.min(jnp.where(eq, sub_iota, NE), axis=0, keepdims=True)
        sel = sub_iota == idx
        tws.append(mk)
        tis.append(idx)
        onehot_sum = onehot_sum + sel.astype(jnp.float32)
        work = jnp.where(sel, -jnp.inf, work)
    tw = jnp.concatenate(tws, axis=0)
    ti = jnp.concatenate(tis, axis=0)
    tw = tw / (jnp.sum(tw, axis=0, keepdims=True) + 1e-20)

    tw_ref[...] = tw.T
    ti_ref[...] = ti.T

    pi_part = jnp.sum(scores.reshape(NE, R // 128, 128), axis=1)
    cnt_part = jnp.sum(onehot_sum.reshape(NE, R // 128, 128), axis=1)

    @pl.when(step == 0)
    def _():
        pi_ref[...] = jnp.zeros_like(pi_ref)
        cnt_ref[...] = jnp.zeros_like(cnt_ref)

    pi_ref[...] += pi_part
    cnt_ref[...] += cnt_part

    @pl.when(step == GRID - 1)
    def _():
        pi_vec = jnp.sum(pi_ref[...], axis=1)
        cnt_vec = jnp.sum(cnt_ref[...], axis=1)
        scale = jnp.float32(ALPHA * NE / (float(T) * K * float(T)))
        aux_ref[0, 0] = jnp.sum(pi_vec * cnt_vec) * scale


def _gate_call(xf, weight):
    return pl.pallas_call(
        _gate_body,
        grid=(GRID,),
        in_specs=[
            pl.BlockSpec((R, D), lambda i: (i, 0)),
            pl.BlockSpec((NE, D), lambda i: (0, 0)),
        ],
        out_specs=[
            pl.BlockSpec((R, K), lambda i: (i, 0)),
            pl.BlockSpec((R, K), lambda i: (i, 0)),
            pl.BlockSpec((1, 1), lambda i: (0, 0)),
        ],
        out_shape=[
            jax.ShapeDtypeStruct((T, K), jnp.float32),
            jax.ShapeDtypeStruct((T, K), jnp.int32),
            jax.ShapeDtypeStruct((1, 1), jnp.float32),
        ],
        scratch_shapes=[
            pltpu.VMEM((NE, 128), jnp.float32),
            pltpu.VMEM((NE, 128), jnp.float32),
        ],
        compiler_params=pltpu.CompilerParams(
            dimension_semantics=("arbitrary",)),
    )(xf, weight)


def kernel(x, weight):
    xf = x.reshape(T, D)
    tw, ti, aux = _gate_call(xf, weight)
    return tw, ti, aux[0, 0]


# speedup vs baseline: 1.7258x; 1.7258x over previous
import jax
import jax.numpy as jnp
from jax.experimental import pallas as pl
from jax.experimental.pallas import tpu as pltpu

NE = 64
K = 8
D = 2048
T = 16384
R = 512
GRID = T // R
ALPHA = 0.001


def _gate_body(x_ref, w_ref, tw_ref, ti_ref, aux_ref, pi_ref, cnt_ref):
    step = pl.program_id(0)
    logits = jax.lax.dot_general(
        w_ref[...], x_ref[...], (((1,), (1,)), ((), ())),
        preferred_element_type=jnp.float32)
    m = jnp.max(logits, axis=0, keepdims=True)
    e = jnp.exp(logits - m)
    s = jnp.sum(e, axis=0, keepdims=True)
    scores = e / s

    sub_iota = jax.lax.broadcasted_iota(jnp.int32, (NE, R), 0)
    work = scores
    tws, tis = [], []
    onehot_sum = jnp.zeros((NE, R), jnp.float32)
    for _ in range(K):
        mk = jnp.max(work, axis=0, keepdims=True)
        eq = work == mk
        idx = jnp.min(jnp.where(eq, sub_iota, NE), axis=0, keepdims=True)
        sel = sub_iota == idx
        tws.append(mk)
        tis.append(idx)
        onehot_sum = onehot_sum + sel.astype(jnp.float32)
        work = jnp.where(sel, -jnp.inf, work)
    tw = jnp.concatenate(tws, axis=0)
    ti = jnp.concatenate(tis, axis=0)
    tw = tw / (jnp.sum(tw, axis=0, keepdims=True) + 1e-20)

    tw_ref[...] = tw.T
    ti_ref[...] = ti.T

    pi_part = jnp.sum(scores.reshape(NE, R // 128, 128), axis=1)
    cnt_part = jnp.sum(onehot_sum.reshape(NE, R // 128, 128), axis=1)

    @pl.when(step == 0)
    def _():
        pi_ref[...] = jnp.zeros_like(pi_ref)
        cnt_ref[...] = jnp.zeros_like(cnt_ref)

    pi_ref[...] += pi_part
    cnt_ref[...] += cnt_part

    @pl.when(step == GRID - 1)
    def _():
        pi_vec = jnp.sum(pi_ref[...], axis=1)
        cnt_vec = jnp.sum(cnt_ref[...], axis=1)
        scale = jnp.float32(ALPHA * NE / (float(T) * K * float(T)))
        aux_ref[...] = jnp.reshape(jnp.sum(pi_vec * cnt_vec) * scale, (1, 1))


def _gate_call(xf, weight):
    return pl.pallas_call(
        _gate_body,
        grid=(GRID,),
        in_specs=[
            pl.BlockSpec((R, D), lambda i: (i, 0)),
            pl.BlockSpec((NE, D), lambda i: (0, 0)),
        ],
        out_specs=[
            pl.BlockSpec((R, K), lambda i: (i, 0)),
            pl.BlockSpec((R, K), lambda i: (i, 0)),
            pl.BlockSpec((1, 1), lambda i: (0, 0)),
        ],
        out_shape=[
            jax.ShapeDtypeStruct((T, K), jnp.float32),
            jax.ShapeDtypeStruct((T, K), jnp.int32),
            jax.ShapeDtypeStruct((1, 1), jnp.float32),
        ],
        scratch_shapes=[
            pltpu.VMEM((NE, 128), jnp.float32),
            pltpu.VMEM((NE, 128), jnp.float32),
        ],
        compiler_params=pltpu.CompilerParams(
            dimension_semantics=("arbitrary",)),
    )(xf, weight)


def kernel(x, weight):
    xf = x.reshape(T, D)
    tw, ti, aux = _gate_call(xf, weight)
    return tw, ti, aux[0, 0]
